# baseline (speedup 1.0000x reference)
import functools

import jax
import jax.numpy as jnp
from jax import lax
from jax.experimental import pallas as pl
from jax.experimental.pallas import tpu as pltpu
from jax.experimental.pallas import tpu_sc as plsc

DIM = 1024
MEM = 512
TPM = 256
TOPK = 8
BATCH = 4
SEQ = 2048

NC, NS, L = 2, 16, 16
NW = NC * NS
CPB = NW // BATCH
CHUNK = TPM // CPB
STEPS = 8
TSUB = CHUNK // STEPS
TSUBD = TSUB * DIM
GPS = MEM * TPM * DIM // TSUBD
_PB = 264


def _router_body(xf_ref, r_ref, idx_ref, w_ref):
    scores = jnp.dot(xf_ref[:, 0, :], r_ref[...],
                     preferred_element_type=jnp.float32)
    iota = lax.broadcasted_iota(jnp.int32, (BATCH, MEM), 1)
    run = scores
    vals, idxs = [], []
    for _ in range(TOPK):
        m = jnp.max(run, axis=1, keepdims=True)
        ik = jnp.min(jnp.where(run == m, iota, MEM), axis=1, keepdims=True)
        vals.append(m)
        idxs.append(ik)
        run = jnp.where(iota == ik, -jnp.inf, run)
    v = jnp.concatenate(vals, axis=1)
    i = jnp.concatenate(idxs, axis=1)
    e = jnp.exp(v - v[:, :1])
    w = e / jnp.sum(e, axis=1, keepdims=True)
    pad = ((0, 0), (0, L - TOPK))
    idx_ref[...] = jnp.pad(i, pad)[:, None, :]
    w_ref[...] = jnp.pad(w, pad)[:, None, :]


def _router(x, router):
    return pl.pallas_call(
        _router_body,
        grid=(1,),
        in_specs=[
            pl.BlockSpec((BATCH, 8, DIM), lambda _: (0, 0, 0)),
            pl.BlockSpec((DIM, MEM), lambda _: (0, 0)),
        ],
        out_specs=[
            pl.BlockSpec((BATCH, 1, L), lambda _: (0, 0, 0)),
            pl.BlockSpec((BATCH, 1, L), lambda _: (0, 0, 0)),
        ],
        out_shape=(
            jax.ShapeDtypeStruct((BATCH, 1, L), jnp.int32),
            jax.ShapeDtypeStruct((BATCH, 1, L), jnp.float32),
        ),
    )(x, router)


def _take16(vec, gidx):
    return vec[gidx]


def _combine_body(mem_hbm, idxp_hbm, wp_hbm, out_hbm,
                  i16_v, w16_v, rows_v, g0, g1, o0, o1,
                  sg0, sg1, so0, so1):
    wid = lax.axis_index("s") * NC + lax.axis_index("c")
    b = wid // CPB
    c = wid % CPB

    pltpu.sync_copy(idxp_hbm.at[b], i16_v)
    pltpu.sync_copy(wp_hbm.at[b], w16_v)
    i16 = i16_v[0, :]
    w16 = w16_v[0, :]
    lane = lax.broadcasted_iota(jnp.int32, (L,), 0)

    wks = [_take16(w16, jnp.full((L,), k, jnp.int32)) for k in range(TOPK)]

    gi = _take16(i16, lane % TOPK)
    for p in range(STEPS // 2):
        v = (gi * (TPM * DIM // TSUBD) + c * (CHUNK * DIM // TSUBD)
             + 2 * p + (lane >> 3))
        rows_v[pl.ds(p * L, L)] = v

    gb = (g0, g1)
    ob = (o0, o1)
    sg = (sg0, sg1)
    so = (so0, so1)
    out_dma = [None, None]
    pending = pltpu.async_copy(
        mem_hbm.at[rows_v.at[pl.ds(0, TOPK)]], g0, sg0)
    for step in range(STEPS):
        nxt = None
        if step + 1 < STEPS:
            nxt = pltpu.async_copy(
                mem_hbm.at[rows_v.at[pl.ds((step + 1) * TOPK, TOPK)]],
                gb[(step + 1) % 2], sg[(step + 1) % 2])
        pending.wait()
        g = gb[step % 2]
        pair = step // 2
        half = step % 2
        o = ob[pair % 2]
        if half == 0 and out_dma[pair % 2] is not None:
            out_dma[pair % 2].wait()

        def cbody(q, _, g=g, o=o, half=half):
            sl = pl.ds(q * L, L)
            a = wks[0] * g[0, sl]
            for k in range(1, TOPK):
                a = a + wks[k] * g[k, sl]
            o[pl.ds(half * TSUBD + q * L, L)] = a
            return 0
        lax.fori_loop(0, TSUBD // L, cbody, 0, unroll=2)

        if half == 1:
            out_dma[pair % 2] = pltpu.async_copy(
                o,
                out_hbm.at[pl.ds(wid * CHUNK * DIM + pair * 2 * TSUBD,
                                 2 * TSUBD)],
                so[pair % 2])
        pending = nxt

    out_dma[0].wait()
    out_dma[1].wait()


def _combine(mem64, idxp, wp):
    mesh = plsc.VectorSubcoreMesh(core_axis_name="c", subcore_axis_name="s")
    f = functools.partial(
        pl.kernel,
        out_type=jax.ShapeDtypeStruct((BATCH * TPM * DIM,), jnp.float32),
        mesh=mesh,
        scratch_types=[
            pltpu.VMEM((1, L), jnp.int32),
            pltpu.VMEM((1, L), jnp.float32),
            pltpu.VMEM((STEPS * TOPK,), jnp.int32),
            pltpu.VMEM((TOPK, TSUBD), jnp.float32),
            pltpu.VMEM((TOPK, TSUBD), jnp.float32),
            pltpu.VMEM((2 * TSUBD,), jnp.float32),
            pltpu.VMEM((2 * TSUBD,), jnp.float32),
            pltpu.SemaphoreType.DMA,
            pltpu.SemaphoreType.DMA,
            pltpu.SemaphoreType.DMA,
            pltpu.SemaphoreType.DMA,
        ],
    )(_combine_body)
    return f(mem64, idxp, wp)


_RB = 256


def _copy_body(x_ref, o_ref):
    o_ref[...] = x_ref[...]


def _copy_tail(x):
    nb = SEQ // _RB - 1
    return pl.pallas_call(
        _copy_body,
        grid=(BATCH, nb),
        in_specs=[pl.BlockSpec((1, _RB, DIM), lambda b, j: (b, j + 1, 0))],
        out_specs=pl.BlockSpec((1, _RB, DIM), lambda b, j: (b, j + 1, 0)),
        out_shape=jax.ShapeDtypeStruct((BATCH, SEQ, DIM), jnp.float32),
    )(x)


def _patch_body(alias_ref, xh_ref, comb_ref, o_ref):
    del alias_ref
    o_ref[0, 0:1, :] = xh_ref[0, 0:1, :]
    o_ref[0, pl.ds(1, TPM), :] = comb_ref[0]
    o_ref[0, pl.ds(TPM + 1, _PB - TPM - 1), :] = \
        xh_ref[0, pl.ds(TPM + 1, _PB - TPM - 1), :]


def _patch(out1, x, comb):
    return pl.pallas_call(
        _patch_body,
        grid=(BATCH,),
        in_specs=[
            pl.BlockSpec(memory_space=pl.ANY),
            pl.BlockSpec((1, _PB, DIM), lambda b: (b, 0, 0)),
            pl.BlockSpec((1, TPM, DIM), lambda b: (b, 0, 0)),
        ],
        out_specs=pl.BlockSpec((1, _PB, DIM), lambda b: (b, 0, 0)),
        out_shape=jax.ShapeDtypeStruct((BATCH, SEQ, DIM), jnp.float32),
        input_output_aliases={0: 0},
    )(out1, x, comb)


def kernel(x, memory_tokens, memory_router):
    idxp, wp = _router(x, memory_router)
    mem64 = memory_tokens.reshape(GPS, TSUBD)
    comb = _combine(mem64, idxp, wp).reshape(BATCH, TPM, DIM)
    out1 = _copy_tail(x)
    return _patch(out1, x, comb)


# speedup vs baseline: 9.2883x; 9.2883x over previous
import functools

import jax
import jax.numpy as jnp
from jax import lax
from jax.experimental import pallas as pl
from jax.experimental.pallas import tpu as pltpu
from jax.experimental.pallas import tpu_sc as plsc

DIM = 1024
MEM = 512
TPM = 256
TOPK = 8
BATCH = 4
SEQ = 2048

NC, NS, L = 2, 16, 16
NW = NC * NS
CHUNK = BATCH * TPM // NW
_PB = 264


def _router_body(xf_ref, r_ref, idx_ref, w_ref):
    scores = jnp.dot(xf_ref[:, 0, :], r_ref[...],
                     preferred_element_type=jnp.float32)
    iota = lax.broadcasted_iota(jnp.int32, (BATCH, MEM), 1)
    run = scores
    vals, idxs = [], []
    for _ in range(TOPK):
        m = jnp.max(run, axis=1, keepdims=True)
        ik = jnp.min(jnp.where(run == m, iota, MEM), axis=1, keepdims=True)
        vals.append(m)
        idxs.append(ik)
        run = jnp.where(iota == ik, -jnp.inf, run)
    v = jnp.concatenate(vals, axis=1)
    i = jnp.concatenate(idxs, axis=1)
    e = jnp.exp(v - v[:, :1])
    w_ref[...] = e / jnp.sum(e, axis=1, keepdims=True)
    idx_ref[...] = i


def _router(x, router):
    return pl.pallas_call(
        _router_body,
        grid=(1,),
        in_specs=[
            pl.BlockSpec((BATCH, 8, DIM), lambda _: (0, 0, 0)),
            pl.BlockSpec((DIM, MEM), lambda _: (0, 0)),
        ],
        out_specs=[
            pl.BlockSpec((BATCH, TOPK), lambda _: (0, 0)),
            pl.BlockSpec((BATCH, TOPK), lambda _: (0, 0)),
        ],
        out_shape=(
            jax.ShapeDtypeStruct((BATCH, TOPK), jnp.int32),
            jax.ShapeDtypeStruct((BATCH, TOPK), jnp.float32),
        ),
    )(x, router)


STEPS = 8
TSUB = CHUNK // STEPS
GROWS = TOPK * TSUB


def _combine_body(mem_hbm, rows_hbm, w_hbm, out_hbm,
                  idx_v, w_v, g0, g1, o0, o1, sg0, sg1, so0, so1):
    wid = lax.axis_index("s") * NC + lax.axis_index("c")
    pltpu.sync_copy(rows_hbm.at[wid], idx_v)
    pltpu.sync_copy(w_hbm.at[wid], w_v)
    wks = [w_v[k, :] for k in range(TOPK)]

    gb = (g0, g1)
    ob = (o0, o1)
    sg = (sg0, sg1)
    so = (so0, so1)
    out_dma = [None, None]
    pending = pltpu.async_copy(mem_hbm.at[idx_v.at[0]], g0, sg0)
    for step in range(STEPS):
        nxt = None
        if step + 1 < STEPS:
            nxt = pltpu.async_copy(mem_hbm.at[idx_v.at[step + 1]],
                                   gb[(step + 1) % 2], sg[(step + 1) % 2])
        pending.wait()
        g = gb[step % 2]
        pair = step // 2
        half = step % 2
        o = ob[pair % 2]
        if half == 0 and out_dma[pair % 2] is not None:
            out_dma[pair % 2].wait()

        for t in range(TSUB):
            def cbody(ci, _, g=g, o=o, t=t, half=half):
                sl = pl.ds(ci * L, L)
                a = wks[0] * g[t, sl]
                for k in range(1, TOPK):
                    a = a + wks[k] * g[k * TSUB + t, sl]
                o[half * TSUB + t, sl] = a
                return 0
            lax.fori_loop(0, DIM // L, cbody, 0, unroll=2)

        if half == 1:
            out_dma[pair % 2] = pltpu.async_copy(
                o,
                out_hbm.at[pl.ds(wid * CHUNK + pair * 2 * TSUB, 2 * TSUB)],
                so[pair % 2])
        pending = nxt

    out_dma[0].wait()
    out_dma[1].wait()


def _combine(mem2d, rows, wsplat):
    mesh = plsc.VectorSubcoreMesh(core_axis_name="c", subcore_axis_name="s")
    f = functools.partial(
        pl.kernel,
        out_type=jax.ShapeDtypeStruct((BATCH * TPM, DIM), jnp.float32),
        mesh=mesh,
        scratch_types=[
            pltpu.VMEM((STEPS, GROWS), jnp.int32),
            pltpu.VMEM((TOPK, L), jnp.float32),
            pltpu.VMEM((GROWS, DIM), jnp.float32),
            pltpu.VMEM((GROWS, DIM), jnp.float32),
            pltpu.VMEM((2 * TSUB, DIM), jnp.float32),
            pltpu.VMEM((2 * TSUB, DIM), jnp.float32),
            pltpu.SemaphoreType.DMA,
            pltpu.SemaphoreType.DMA,
            pltpu.SemaphoreType.DMA,
            pltpu.SemaphoreType.DMA,
        ],
    )(_combine_body)
    return f(mem2d, rows, wsplat)


_RB = 256


def _copy_body(x_ref, o_ref):
    o_ref[...] = x_ref[...]


def _copy_tail(x):
    nb = SEQ // _RB - 1
    return pl.pallas_call(
        _copy_body,
        grid=(BATCH, nb),
        in_specs=[pl.BlockSpec((1, _RB, DIM), lambda b, j: (b, j + 1, 0))],
        out_specs=pl.BlockSpec((1, _RB, DIM), lambda b, j: (b, j + 1, 0)),
        out_shape=jax.ShapeDtypeStruct((BATCH, SEQ, DIM), jnp.float32),
    )(x)


def _patch_body(alias_ref, xh_ref, comb_ref, o_ref):
    del alias_ref
    o_ref[0, 0:1, :] = xh_ref[0, 0:1, :]
    o_ref[0, pl.ds(1, TPM), :] = comb_ref[0]
    o_ref[0, pl.ds(TPM + 1, _PB - TPM - 1), :] = \
        xh_ref[0, pl.ds(TPM + 1, _PB - TPM - 1), :]


def _patch(out1, xh, comb):
    return pl.pallas_call(
        _patch_body,
        grid=(BATCH,),
        in_specs=[
            pl.BlockSpec(memory_space=pl.ANY),
            pl.BlockSpec((1, _PB, DIM), lambda b: (b, 0, 0)),
            pl.BlockSpec((1, TPM, DIM), lambda b: (b, 0, 0)),
        ],
        out_specs=pl.BlockSpec((1, _PB, DIM), lambda b: (b, 0, 0)),
        out_shape=jax.ShapeDtypeStruct((BATCH, SEQ, DIM), jnp.float32),
        input_output_aliases={0: 0},
    )(out1, xh, comb)


def kernel(x, memory_tokens, memory_router):
    idx, w = _router(x, memory_router)

    chunks = jnp.arange(NW // BATCH, dtype=jnp.int32)
    steps = jnp.arange(STEPS, dtype=jnp.int32)
    toks = jnp.arange(TSUB, dtype=jnp.int32)
    rows = (idx[:, None, None, :, None] * TPM
            + chunks[None, :, None, None, None] * CHUNK
            + steps[None, None, :, None, None] * TSUB
            + toks[None, None, None, None, :])
    rows = rows.reshape(NW, STEPS, GROWS)
    wsplat = jnp.broadcast_to(w[:, None, :, None],
                              (BATCH, NW // BATCH, TOPK, L))
    wsplat = wsplat.reshape(NW, TOPK, L)

    mem2d = memory_tokens.reshape(MEM * TPM, DIM)
    comb = _combine(mem2d, rows, wsplat).reshape(BATCH, TPM, DIM)
    out1 = _copy_tail(x)
    return _patch(out1, x, comb)


# speedup vs baseline: 9.3776x; 1.0096x over previous
import functools

import jax
import jax.numpy as jnp
from jax import lax
from jax.experimental import pallas as pl
from jax.experimental.pallas import tpu as pltpu
from jax.experimental.pallas import tpu_sc as plsc

DIM = 1024
MEM = 512
TPM = 256
TOPK = 8
BATCH = 4
SEQ = 2048

NC, NS, L = 2, 16, 16
NW = NC * NS
CPB = NW // BATCH
CHUNK = BATCH * TPM // NW
_PB = 264


def _router_body(xf_ref, r_ref, idx_ref, w_ref):
    scores = jnp.dot(xf_ref[:, 0, :], r_ref[...],
                     preferred_element_type=jnp.float32)
    iota = lax.broadcasted_iota(jnp.int32, (BATCH, MEM), 1)
    run = scores
    vals, idxs = [], []
    for _ in range(TOPK):
        m = jnp.max(run, axis=1, keepdims=True)
        ik = jnp.min(jnp.where(run == m, iota, MEM), axis=1, keepdims=True)
        vals.append(m)
        idxs.append(ik)
        run = jnp.where(iota == ik, -jnp.inf, run)
    v = jnp.concatenate(vals, axis=1)
    i = jnp.concatenate(idxs, axis=1)
    e = jnp.exp(v - v[:, :1])
    w = e / jnp.sum(e, axis=1, keepdims=True)
    pad = ((0, 0), (0, L - TOPK))
    idx_ref[...] = jnp.pad(i, pad)[:, None, :]
    w_ref[...] = jnp.pad(w, pad)[:, None, :]


def _router(x, router):
    return pl.pallas_call(
        _router_body,
        grid=(1,),
        in_specs=[
            pl.BlockSpec((BATCH, 8, DIM), lambda _: (0, 0, 0)),
            pl.BlockSpec((DIM, MEM), lambda _: (0, 0)),
        ],
        out_specs=[
            pl.BlockSpec((BATCH, 1, L), lambda _: (0, 0, 0)),
            pl.BlockSpec((BATCH, 1, L), lambda _: (0, 0, 0)),
        ],
        out_shape=(
            jax.ShapeDtypeStruct((BATCH, 1, L), jnp.int32),
            jax.ShapeDtypeStruct((BATCH, 1, L), jnp.float32),
        ),
    )(x, router)


STEPS = 8
TSUB = CHUNK // STEPS
GROWS = TOPK * TSUB


def _take16(vec, gidx):
    return vec[gidx]


def _combine_body(mem_hbm, idxp_hbm, wp_hbm, out_hbm,
                  i16_v, w16_v, idx_v, g0, g1, o0, o1, sg0, sg1, so0, so1):
    wid = lax.axis_index("s") * NC + lax.axis_index("c")
    b = wid // CPB
    c = wid % CPB
    pltpu.sync_copy(idxp_hbm.at[b], i16_v)
    pltpu.sync_copy(wp_hbm.at[b], w16_v)
    w16 = w16_v[0, :]
    i16 = i16_v[0, :]
    lane = lax.broadcasted_iota(jnp.int32, (L,), 0)

    wks = [_take16(w16, jnp.full((L,), k, jnp.int32)) for k in range(TOPK)]

    for half in range(2):
        gi = _take16(i16, (half * L + lane) >> 2)
        base = gi * TPM + c * CHUNK + (lane & 3)
        for step in range(STEPS):
            idx_v[step, pl.ds(half * L, L)] = base + step * TSUB

    gb = (g0, g1)
    ob = (o0, o1)
    sg = (sg0, sg1)
    so = (so0, so1)
    out_dma = [None, None]
    pending = pltpu.async_copy(mem_hbm.at[idx_v.at[0]], g0, sg0)
    for step in range(STEPS):
        nxt = None
        if step + 1 < STEPS:
            nxt = pltpu.async_copy(mem_hbm.at[idx_v.at[step + 1]],
                                   gb[(step + 1) % 2], sg[(step + 1) % 2])
        pending.wait()
        g = gb[step % 2]
        pair = step // 2
        half = step % 2
        o = ob[pair % 2]
        if half == 0 and out_dma[pair % 2] is not None:
            out_dma[pair % 2].wait()

        for t in range(TSUB):
            def cbody(ci, _, g=g, o=o, t=t, half=half):
                sl = pl.ds(ci * L, L)
                a = wks[0] * g[t, sl]
                for k in range(1, TOPK):
                    a = a + wks[k] * g[k * TSUB + t, sl]
                o[half * TSUB + t, sl] = a
                return 0
            lax.fori_loop(0, DIM // L, cbody, 0, unroll=2)

        if half == 1:
            out_dma[pair % 2] = pltpu.async_copy(
                o,
                out_hbm.at[pl.ds(wid * CHUNK + pair * 2 * TSUB, 2 * TSUB)],
                so[pair % 2])
        pending = nxt

    out_dma[0].wait()
    out_dma[1].wait()


def _combine(mem2d, idxp, wp):
    mesh = plsc.VectorSubcoreMesh(core_axis_name="c", subcore_axis_name="s")
    f = functools.partial(
        pl.kernel,
        out_type=jax.ShapeDtypeStruct((BATCH * TPM, DIM), jnp.float32),
        mesh=mesh,
        scratch_types=[
            pltpu.VMEM((1, L), jnp.int32),
            pltpu.VMEM((1, L), jnp.float32),
            pltpu.VMEM((STEPS, GROWS), jnp.int32),
            pltpu.VMEM((GROWS, DIM), jnp.float32),
            pltpu.VMEM((GROWS, DIM), jnp.float32),
            pltpu.VMEM((2 * TSUB, DIM), jnp.float32),
            pltpu.VMEM((2 * TSUB, DIM), jnp.float32),
            pltpu.SemaphoreType.DMA,
            pltpu.SemaphoreType.DMA,
            pltpu.SemaphoreType.DMA,
            pltpu.SemaphoreType.DMA,
        ],
    )(_combine_body)
    return f(mem2d, idxp, wp)


_RB = 256


def _copy_body(x_ref, o_ref):
    o_ref[...] = x_ref[...]


def _copy_tail(x):
    nb = SEQ // _RB - 1
    return pl.pallas_call(
        _copy_body,
        grid=(BATCH, nb),
        in_specs=[pl.BlockSpec((1, _RB, DIM), lambda b, j: (b, j + 1, 0))],
        out_specs=pl.BlockSpec((1, _RB, DIM), lambda b, j: (b, j + 1, 0)),
        out_shape=jax.ShapeDtypeStruct((BATCH, SEQ, DIM), jnp.float32),
    )(x)


def _patch_body(alias_ref, xh_ref, comb_ref, o_ref):
    del alias_ref
    o_ref[0, 0:1, :] = xh_ref[0, 0:1, :]
    o_ref[0, pl.ds(1, TPM), :] = comb_ref[0]
    o_ref[0, pl.ds(TPM + 1, _PB - TPM - 1), :] = \
        xh_ref[0, pl.ds(TPM + 1, _PB - TPM - 1), :]


def _patch(out1, xh, comb):
    return pl.pallas_call(
        _patch_body,
        grid=(BATCH,),
        in_specs=[
            pl.BlockSpec(memory_space=pl.ANY),
            pl.BlockSpec((1, _PB, DIM), lambda b: (b, 0, 0)),
            pl.BlockSpec((1, TPM, DIM), lambda b: (b, 0, 0)),
        ],
        out_specs=pl.BlockSpec((1, _PB, DIM), lambda b: (b, 0, 0)),
        out_shape=jax.ShapeDtypeStruct((BATCH, SEQ, DIM), jnp.float32),
        input_output_aliases={0: 0},
    )(out1, xh, comb)


def kernel(x, memory_tokens, memory_router):
    idxp, wp = _router(x, memory_router)
    mem2d = memory_tokens.reshape(MEM * TPM, DIM)
    comb = _combine(mem2d, idxp, wp).reshape(BATCH, TPM, DIM)
    out1 = _copy_tail(x)
    return _patch(out1, x, comb)
